# Initial kernel scaffold; baseline (speedup 1.0000x reference)
import functools
import math

import jax
import jax.numpy as jnp
import numpy as np
from jax import lax
from jax.experimental import pallas as pl
from jax.experimental.pallas import tpu as pltpu
from jax.experimental.pallas import tpu_sc as plsc

F32 = jnp.float32
HIDDEN = 128
NFILT = 128
NGAUSS = 50
NINT = 3
CUTOFF = 5.0
NSEG = 64
N_NODES = 10000
N_EDGES = 320000
SHIFT = math.log(2.0)

_OFFSETS = np.linspace(0.0, CUTOFF, NGAUSS).astype(np.float32)
GAMMA = float(0.5 / (_OFFSETS[1] - _OFFSETS[0]) ** 2)
PI_C = float(math.pi / CUTOFF)

NC = 2
NS = 16
NW = NC * NS
EW = N_EDGES // NW
KM = 125
CM = EW // KM
KF = 80
CF = EW // KF
NPT = N_NODES // NS

BE = 1280
GE = N_EDGES // BE
BN = 1000
GN = N_NODES // BN

_mesh = plsc.VectorSubcoreMesh(core_axis_name="c", subcore_axis_name="s")


def _ssp(x):
    return jnp.maximum(x, 0.0) + jnp.log1p(jnp.exp(-jnp.abs(x))) - SHIFT


def _sig(x):
    return jax.nn.sigmoid(x)


def _dotT(a, b):
    return lax.dot_general(a, b, (((1,), (1,)), ((), ())),
                           preferred_element_type=F32)


def _dot(a, b):
    return jnp.dot(a, b, preferred_element_type=F32)


def _node0_body(an_ref, emb_ref, lin1_ref, out_ref):
    an = an_ref[...]
    ids = lax.broadcasted_iota(jnp.int32, (BN, 100), 1)
    oh = (an == ids).astype(F32)
    h0 = _dot(oh, emb_ref[...])
    out_ref[...] = _dot(h0, lin1_ref[...])


def _node0(an2, emb, lin1):
    return pl.pallas_call(
        _node0_body,
        grid=(GN,),
        in_specs=[
            pl.BlockSpec((BN, 1), lambda i: (i, 0)),
            pl.BlockSpec((100, HIDDEN), lambda i: (0, 0)),
            pl.BlockSpec((HIDDEN, HIDDEN), lambda i: (0, 0)),
        ],
        out_specs=pl.BlockSpec((BN, HIDDEN), lambda i: (i, 0)),
        out_shape=jax.ShapeDtypeStruct((N_NODES, HIDDEN), F32),
    )(an2, emb, lin1)


def _edge_mlp_body(ss_ref, w1_ref, b1_ref, w2_ref, b2_ref, out_ref):
    ew = jnp.sqrt(ss_ref[...] + 1e-12)
    off = jnp.asarray(_OFFSETS)[None, :]
    ea = jnp.exp(-GAMMA * (ew - off) ** 2)
    a1 = _dot(ea, w1_ref[...]) + b1_ref[...]
    sp = _ssp(a1)
    filt = _dot(sp, w2_ref[...]) + b2_ref[...]
    cth = 0.5 * (jnp.cos(ew * PI_C) + 1.0)
    out_ref[...] = cth * filt


def _edge_mlp(ss2, w1, b1, w2, b2):
    return pl.pallas_call(
        _edge_mlp_body,
        grid=(GE,),
        in_specs=[
            pl.BlockSpec((BE, 1), lambda i: (i, 0)),
            pl.BlockSpec((NGAUSS, NFILT), lambda i: (0, 0)),
            pl.BlockSpec((1, NFILT), lambda i: (0, 0)),
            pl.BlockSpec((NFILT, NFILT), lambda i: (0, 0)),
            pl.BlockSpec((1, NFILT), lambda i: (0, 0)),
        ],
        out_specs=pl.BlockSpec((BE, NFILT), lambda i: (i, 0)),
        out_shape=jax.ShapeDtypeStruct((N_EDGES, NFILT), F32),
    )(ss2, w1, b1, w2, b2)


def _node_t_body(parts_ref, lin2_ref, b2_ref, linw_ref, linb_ref,
                 t_ref, h_ref):
    agg = parts_ref[0] + parts_ref[1]
    t = _dot(agg, lin2_ref[...]) + b2_ref[...]
    t_ref[...] = t
    h_ref[...] = _dot(_ssp(t), linw_ref[...]) + linb_ref[...]


def _node_t(parts, lin2, b2, linw, linb):
    return pl.pallas_call(
        _node_t_body,
        grid=(GN,),
        in_specs=[
            pl.BlockSpec((2, BN, HIDDEN), lambda i: (0, i, 0)),
            pl.BlockSpec((HIDDEN, HIDDEN), lambda i: (0, 0)),
            pl.BlockSpec((1, HIDDEN), lambda i: (0, 0)),
            pl.BlockSpec((HIDDEN, HIDDEN), lambda i: (0, 0)),
            pl.BlockSpec((1, HIDDEN), lambda i: (0, 0)),
        ],
        out_specs=[
            pl.BlockSpec((BN, HIDDEN), lambda i: (i, 0)),
            pl.BlockSpec((BN, HIDDEN), lambda i: (i, 0)),
        ],
        out_shape=[
            jax.ShapeDtypeStruct((N_NODES, HIDDEN), F32),
            jax.ShapeDtypeStruct((N_NODES, HIDDEN), F32),
        ],
    )(parts, lin2, b2, linw, linb)


def _mm_body(x_ref, w_ref, o_ref):
    o_ref[...] = _dot(x_ref[...], w_ref[...])


def _mm(x, w):
    return pl.pallas_call(
        _mm_body,
        grid=(GN,),
        in_specs=[
            pl.BlockSpec((BN, HIDDEN), lambda i: (i, 0)),
            pl.BlockSpec((HIDDEN, HIDDEN), lambda i: (0, 0)),
        ],
        out_specs=pl.BlockSpec((BN, HIDDEN), lambda i: (i, 0)),
        out_shape=jax.ShapeDtypeStruct((N_NODES, HIDDEN), F32),
    )(x, w)


def _node_bwd_body(g_ref, t_ref, linw_ref, lin2_ref, o_ref):
    gu = _dotT(g_ref[...], linw_ref[...])
    gt = gu * _sig(t_ref[...])
    o_ref[...] = _dotT(gt, lin2_ref[...])


def _node_bwd(g, t, linw, lin2):
    return pl.pallas_call(
        _node_bwd_body,
        grid=(GN,),
        in_specs=[
            pl.BlockSpec((BN, HIDDEN), lambda i: (i, 0)),
            pl.BlockSpec((BN, HIDDEN), lambda i: (i, 0)),
            pl.BlockSpec((HIDDEN, HIDDEN), lambda i: (0, 0)),
            pl.BlockSpec((HIDDEN, HIDDEN), lambda i: (0, 0)),
        ],
        out_specs=pl.BlockSpec((BN, HIDDEN), lambda i: (i, 0)),
        out_shape=jax.ShapeDtypeStruct((N_NODES, HIDDEN), F32),
    )(g, t, linw, lin2)


def _node_bwd_h1_body(parts_ref, lin1_ref, o_ref):
    s = parts_ref[0] + parts_ref[1]
    o_ref[...] = _dotT(s, lin1_ref[...])


def _node_bwd_h1(parts, lin1):
    return pl.pallas_call(
        _node_bwd_h1_body,
        grid=(GN,),
        in_specs=[
            pl.BlockSpec((2, BN, HIDDEN), lambda i: (0, i, 0)),
            pl.BlockSpec((HIDDEN, HIDDEN), lambda i: (0, 0)),
        ],
        out_specs=pl.BlockSpec((BN, HIDDEN), lambda i: (i, 0)),
        out_shape=jax.ShapeDtypeStruct((N_NODES, HIDDEN), F32),
    )(parts, lin1)


def _edge_bwd_body(ss_ref, gW_ref, w1_ref, b1_ref, w2_ref, b2_ref, acc_ref,
                   out_ref, *, final):
    ew = jnp.sqrt(ss_ref[...] + 1e-12)
    off = jnp.asarray(_OFFSETS)[None, :]
    ea = jnp.exp(-GAMMA * (ew - off) ** 2)
    a1 = _dot(ea, w1_ref[...]) + b1_ref[...]
    sp = _ssp(a1)
    filt = _dot(sp, w2_ref[...]) + b2_ref[...]
    cth = 0.5 * (jnp.cos(ew * PI_C) + 1.0)
    gW = gW_ref[...]
    g_c = jnp.sum(gW * filt, axis=1, keepdims=True)
    g_filt = cth * gW
    g_sp = _dotT(g_filt, w2_ref[...])
    g_a1 = g_sp * _sig(a1)
    g_ea = _dotT(g_a1, w1_ref[...])
    dea = ea * (-2.0 * GAMMA) * (ew - off)
    contrib = (jnp.sum(g_ea * dea, axis=1, keepdims=True)
               + g_c * (-0.5 * PI_C) * jnp.sin(ew * PI_C))
    tot = acc_ref[...] + contrib
    if final:
        tot = tot / ew
    out_ref[...] = tot


def _edge_bwd(ss2, gW, w1, b1, w2, b2, acc, final):
    return pl.pallas_call(
        functools.partial(_edge_bwd_body, final=final),
        grid=(GE,),
        in_specs=[
            pl.BlockSpec((BE, 1), lambda i: (i, 0)),
            pl.BlockSpec((BE, NFILT), lambda i: (i, 0)),
            pl.BlockSpec((NGAUSS, NFILT), lambda i: (0, 0)),
            pl.BlockSpec((1, NFILT), lambda i: (0, 0)),
            pl.BlockSpec((NFILT, NFILT), lambda i: (0, 0)),
            pl.BlockSpec((1, NFILT), lambda i: (0, 0)),
            pl.BlockSpec((BE, 1), lambda i: (i, 0)),
        ],
        out_specs=pl.BlockSpec((BE, 1), lambda i: (i, 0)),
        out_shape=jax.ShapeDtypeStruct((N_EDGES, 1), F32),
    )(ss2, gW, w1, b1, w2, b2, acc)


def _readout_body(h_ref, seg_ref, ew_ref, eb_ref, sw_ref, sb_ref,
                  en_ref, st_ref):
    i = pl.program_id(0)
    h = h_ref[...]
    e = _dot(h, ew_ref[...]) + eb_ref[...]
    hs = _dot(h, sw_ref[...]) + sb_ref[...]
    ids = lax.broadcasted_iota(jnp.int32, (BN, NSEG), 1)
    oh = (seg_ref[...] == ids).astype(F32)
    en_c = lax.dot_general(oh, e, (((0,), (0,)), ((), ())),
                           preferred_element_type=F32)
    st_c = lax.dot_general(oh, hs, (((0,), (0,)), ((), ())),
                           preferred_element_type=F32)

    @pl.when(i == 0)
    def _():
        en_ref[...] = en_c
        st_ref[...] = st_c

    @pl.when(i != 0)
    def _():
        en_ref[...] += en_c
        st_ref[...] += st_c


def _readout(h, seg2, energy_w, energy_b, stress_w, stress_b):
    return pl.pallas_call(
        _readout_body,
        grid=(GN,),
        in_specs=[
            pl.BlockSpec((BN, HIDDEN), lambda i: (i, 0)),
            pl.BlockSpec((BN, 1), lambda i: (i, 0)),
            pl.BlockSpec((HIDDEN, 1), lambda i: (0, 0)),
            pl.BlockSpec((1, 1), lambda i: (0, 0)),
            pl.BlockSpec((HIDDEN, 6), lambda i: (0, 0)),
            pl.BlockSpec((1, 6), lambda i: (0, 0)),
        ],
        out_specs=[
            pl.BlockSpec((NSEG, 1), lambda i: (0, 0)),
            pl.BlockSpec((NSEG, 6), lambda i: (0, 0)),
        ],
        out_shape=[
            jax.ShapeDtypeStruct((NSEG, 1), F32),
            jax.ShapeDtypeStruct((NSEG, 6), F32),
        ],
    )(h, seg2, energy_w, energy_b, stress_w, stress_b)


def _force_fin_body(parts_ref, o_ref):
    s = parts_ref[0] + parts_ref[1]
    o_ref[...] = -s[:, :3]


def _force_fin(parts):
    return pl.pallas_call(
        _force_fin_body,
        grid=(GN,),
        in_specs=[pl.BlockSpec((2, BN, 16), lambda i: (0, i, 0))],
        out_specs=pl.BlockSpec((BN, 3), lambda i: (i, 0)),
        out_shape=jax.ShapeDtypeStruct((N_NODES, 3), F32),
    )(parts)


def _geom_sc_body(px_hbm, py_hbm, pz_hbm, rowf_hbm, colf_hbm, out_hbm,
                  pxb, pyb, pzb, rbuf, cbuf, obuf):
    c = lax.axis_index("c")
    s = lax.axis_index("s")
    wid = c * NS + s
    pltpu.sync_copy(px_hbm, pxb)
    pltpu.sync_copy(py_hbm, pyb)
    pltpu.sync_copy(pz_hbm, pzb)
    pltpu.sync_copy(rowf_hbm.at[wid], rbuf)
    pltpu.sync_copy(colf_hbm.at[wid], cbuf)

    def chunk(ci, carry):
        def grp(g, carry2):
            ri = rbuf[ci, pl.ds(g * 16, 16)]
            cj = cbuf[ci, pl.ds(g * 16, 16)]
            ax = plsc.load_gather(pxb, [ri]) - plsc.load_gather(pxb, [cj])
            ay = plsc.load_gather(pyb, [ri]) - plsc.load_gather(pyb, [cj])
            az = plsc.load_gather(pzb, [ri]) - plsc.load_gather(pzb, [cj])
            obuf[pl.ds(ci * KF + g * 16, 16)] = ax * ax + ay * ay + az * az
            return carry2
        return lax.fori_loop(0, KF // 16, grp, carry)

    lax.fori_loop(0, CF, chunk, 0)
    pltpu.sync_copy(obuf, out_hbm.at[pl.ds(wid * EW, EW)])


@functools.partial(
    pl.kernel,
    out_type=jax.ShapeDtypeStruct((N_EDGES,), F32),
    mesh=_mesh,
    scratch_types=[
        pltpu.VMEM((N_NODES,), F32),
        pltpu.VMEM((N_NODES,), F32),
        pltpu.VMEM((N_NODES,), F32),
        pltpu.VMEM((CF, KF), jnp.int32),
        pltpu.VMEM((CF, KF), jnp.int32),
        pltpu.VMEM((EW,), F32),
    ],
)
def _geom_sc(px, py, pz, rowf, colf, out, pxb, pyb, pzb, rbuf, cbuf, obuf):
    _geom_sc_body(px, py, pz, rowf, colf, out, pxb, pyb, pzb, rbuf, cbuf,
                  obuf)


def _msg_fwd_sc_body(W_hbm, h1_hbm, rowm_hbm, colm_hbm, z_hbm, out_hbm,
                     rbuf, cbuf, wbuf, hbuf, mbuf, agg_sp, sem):
    c = lax.axis_index("c")
    s = lax.axis_index("s")
    wid = c * NS + s
    pltpu.sync_copy(z_hbm, agg_sp.at[pl.ds(s * NPT, NPT)])
    plsc.subcore_barrier()
    pltpu.sync_copy(rowm_hbm.at[wid], rbuf)
    pltpu.sync_copy(colm_hbm.at[wid], cbuf)
    base = wid * EW

    def chunk(j, carry):
        d = pltpu.async_copy(h1_hbm.at[cbuf.at[j]], hbuf, sem)
        pltpu.sync_copy(W_hbm.at[pl.ds(base + j * KM, KM)], wbuf)
        d.wait()

        def rowloop(r, carry2):
            for f in range(8):
                sl = pl.ds(f * 16, 16)
                mbuf[r, sl] = wbuf[r, sl] * hbuf[r, sl]
            return carry2

        lax.fori_loop(0, KM, rowloop, 0)
        pltpu.sync_copy(mbuf, agg_sp.at[rbuf.at[j]], add=True)
        return carry

    lax.fori_loop(0, CM, chunk, 0)
    plsc.subcore_barrier()
    pltpu.sync_copy(agg_sp.at[pl.ds(s * NPT, NPT)],
                    out_hbm.at[c, pl.ds(s * NPT, NPT)])


@functools.partial(
    pl.kernel,
    out_type=jax.ShapeDtypeStruct((NC, N_NODES, HIDDEN), F32),
    mesh=_mesh,
    scratch_types=[
        pltpu.VMEM((CM, KM), jnp.int32),
        pltpu.VMEM((CM, KM), jnp.int32),
        pltpu.VMEM((KM, HIDDEN), F32),
        pltpu.VMEM((KM, HIDDEN), F32),
        pltpu.VMEM((KM, HIDDEN), F32),
        pltpu.VMEM_SHARED((N_NODES, HIDDEN), F32),
        pltpu.SemaphoreType.DMA,
    ],
)
def _msg_fwd_sc(W, h1, rowm, colm, z, out, rbuf, cbuf, wbuf, hbuf, mbuf,
                agg_sp, sem):
    _msg_fwd_sc_body(W, h1, rowm, colm, z, out, rbuf, cbuf, wbuf, hbuf,
                     mbuf, agg_sp, sem)


def _msg_bwd_sc_body(W_hbm, h1_hbm, gagg_hbm, rowm_hbm, colm_hbm, z_hbm,
                     gw_hbm, out_hbm, rbuf, cbuf, wbuf, hbuf, gbuf, obuf,
                     mbuf, gh1_sp, sem):
    c = lax.axis_index("c")
    s = lax.axis_index("s")
    wid = c * NS + s
    pltpu.sync_copy(z_hbm, gh1_sp.at[pl.ds(s * NPT, NPT)])
    plsc.subcore_barrier()
    pltpu.sync_copy(rowm_hbm.at[wid], rbuf)
    pltpu.sync_copy(colm_hbm.at[wid], cbuf)
    base = wid * EW

    def chunk(j, carry):
        d1 = pltpu.async_copy(gagg_hbm.at[rbuf.at[j]], gbuf, sem)
        d2 = pltpu.async_copy(h1_hbm.at[cbuf.at[j]], hbuf, sem)
        pltpu.sync_copy(W_hbm.at[pl.ds(base + j * KM, KM)], wbuf)
        d1.wait()
        d2.wait()

        def rowloop(r, carry2):
            for f in range(8):
                sl = pl.ds(f * 16, 16)
                g16 = gbuf[r, sl]
                obuf[r, sl] = g16 * hbuf[r, sl]
                mbuf[r, sl] = g16 * wbuf[r, sl]
            return carry2

        lax.fori_loop(0, KM, rowloop, 0)
        pltpu.sync_copy(obuf, gw_hbm.at[pl.ds(base + j * KM, KM)])
        pltpu.sync_copy(mbuf, gh1_sp.at[cbuf.at[j]], add=True)
        return carry

    lax.fori_loop(0, CM, chunk, 0)
    plsc.subcore_barrier()
    pltpu.sync_copy(gh1_sp.at[pl.ds(s * NPT, NPT)],
                    out_hbm.at[c, pl.ds(s * NPT, NPT)])


@functools.partial(
    pl.kernel,
    out_type=(
        jax.ShapeDtypeStruct((N_EDGES, NFILT), F32),
        jax.ShapeDtypeStruct((NC, N_NODES, HIDDEN), F32),
    ),
    mesh=_mesh,
    scratch_types=[
        pltpu.VMEM((CM, KM), jnp.int32),
        pltpu.VMEM((CM, KM), jnp.int32),
        pltpu.VMEM((KM, HIDDEN), F32),
        pltpu.VMEM((KM, HIDDEN), F32),
        pltpu.VMEM((KM, HIDDEN), F32),
        pltpu.VMEM((KM, HIDDEN), F32),
        pltpu.VMEM((KM, HIDDEN), F32),
        pltpu.VMEM_SHARED((N_NODES, HIDDEN), F32),
        pltpu.SemaphoreType.DMA,
    ],
)
def _msg_bwd_sc(W, h1, gagg, rowm, colm, z, gw_out, gh1_out, rbuf, cbuf,
                wbuf, hbuf, gbuf, obuf, mbuf, gh1_sp, sem):
    _msg_bwd_sc_body(W, h1, gagg, rowm, colm, z, gw_out, gh1_out, rbuf,
                     cbuf, wbuf, hbuf, gbuf, obuf, mbuf, gh1_sp, sem)


def _force_sc_body(px_hbm, py_hbm, pz_hbm, rowf_hbm, colf_hbm, gs_hbm,
                   z_hbm, out_hbm, pxb, pyb, pzb, rbuf, cbuf, gsb, sbuf,
                   nbuf, fsp):
    c = lax.axis_index("c")
    s = lax.axis_index("s")
    wid = c * NS + s
    pltpu.sync_copy(z_hbm, fsp.at[pl.ds(s * NPT, NPT)])
    plsc.subcore_barrier()
    pltpu.sync_copy(px_hbm, pxb)
    pltpu.sync_copy(py_hbm, pyb)
    pltpu.sync_copy(pz_hbm, pzb)
    pltpu.sync_copy(rowf_hbm.at[wid], rbuf)
    pltpu.sync_copy(colf_hbm.at[wid], cbuf)
    pltpu.sync_copy(gs_hbm.at[wid], gsb)

    zero16 = jnp.zeros((16,), F32)

    def zrow(r, carry):
        sbuf[r, pl.ds(0, 16)] = zero16
        nbuf[r, pl.ds(0, 16)] = zero16
        return carry

    lax.fori_loop(0, KF, zrow, 0)

    lanes = lax.iota(jnp.int32, 16)
    col0 = jnp.zeros((16,), jnp.int32)
    col1 = col0 + 1
    col2 = col0 + 2

    def chunk(ci, carry):
        def grp(g, carry2):
            ri = rbuf[ci, pl.ds(g * 16, 16)]
            cj = cbuf[ci, pl.ds(g * 16, 16)]
            gsv = gsb[ci, pl.ds(g * 16, 16)]
            dx = (plsc.load_gather(pxb, [ri])
                  - plsc.load_gather(pxb, [cj])) * gsv
            dy = (plsc.load_gather(pyb, [ri])
                  - plsc.load_gather(pyb, [cj])) * gsv
            dz = (plsc.load_gather(pzb, [ri])
                  - plsc.load_gather(pzb, [cj])) * gsv
            rows = lanes + g * 16
            plsc.store_scatter(sbuf, [rows, col0], dx)
            plsc.store_scatter(sbuf, [rows, col1], dy)
            plsc.store_scatter(sbuf, [rows, col2], dz)
            plsc.store_scatter(nbuf, [rows, col0], -dx)
            plsc.store_scatter(nbuf, [rows, col1], -dy)
            plsc.store_scatter(nbuf, [rows, col2], -dz)
            return carry2

        lax.fori_loop(0, KF // 16, grp, 0)
        pltpu.sync_copy(sbuf, fsp.at[rbuf.at[ci]], add=True)
        pltpu.sync_copy(nbuf, fsp.at[cbuf.at[ci]], add=True)
        return carry

    lax.fori_loop(0, CF, chunk, 0)
    plsc.subcore_barrier()
    pltpu.sync_copy(fsp.at[pl.ds(s * NPT, NPT)],
                    out_hbm.at[c, pl.ds(s * NPT, NPT)])


@functools.partial(
    pl.kernel,
    out_type=jax.ShapeDtypeStruct((NC, N_NODES, 16), F32),
    mesh=_mesh,
    scratch_types=[
        pltpu.VMEM((N_NODES,), F32),
        pltpu.VMEM((N_NODES,), F32),
        pltpu.VMEM((N_NODES,), F32),
        pltpu.VMEM((CF, KF), jnp.int32),
        pltpu.VMEM((CF, KF), jnp.int32),
        pltpu.VMEM((CF, KF), F32),
        pltpu.VMEM((KF, 16), F32),
        pltpu.VMEM((KF, 16), F32),
        pltpu.VMEM_SHARED((N_NODES, 16), F32),
    ],
)
def _force_sc(px, py, pz, rowf, colf, gs, z, out, pxb, pyb, pzb, rbuf,
              cbuf, gsb, sbuf, nbuf, fsp):
    _force_sc_body(px, py, pz, rowf, colf, gs, z, out, pxb, pyb, pzb,
                   rbuf, cbuf, gsb, sbuf, nbuf, fsp)


def kernel(atomic_numbers, positions, edge_index, structure_index, params):
    inter = params['interactions']
    row = edge_index[0].astype(jnp.int32)
    col = edge_index[1].astype(jnp.int32)
    row_m = row.reshape(NW, CM, KM)
    col_m = col.reshape(NW, CM, KM)
    row_f = row.reshape(NW, CF, KF)
    col_f = col.reshape(NW, CF, KF)
    px = positions[:, 0]
    py = positions[:, 1]
    pz = positions[:, 2]
    an2 = atomic_numbers.astype(jnp.int32).reshape(N_NODES, 1)
    seg2 = structure_index.astype(jnp.int32).reshape(N_NODES, 1)
    z128 = jnp.zeros((NPT, HIDDEN), F32)
    z16 = jnp.zeros((NPT, 16), F32)

    b1 = [p['mlp_b1'].reshape(1, NFILT) for p in inter]
    b2 = [p['mlp_b2'].reshape(1, NFILT) for p in inter]
    l2b = [p['lin2_b'].reshape(1, HIDDEN) for p in inter]
    lnb = [p['lin_b'].reshape(1, HIDDEN) for p in inter]

    sumsq = _geom_sc(px, py, pz, row_f, col_f)
    ss2 = sumsq.reshape(N_EDGES, 1)

    h1 = _node0(an2, params['emb'], inter[0]['lin1_w'])
    Ws = []
    h1s = []
    ts = []
    h = None
    for i in range(NINT):
        p = inter[i]
        W = _edge_mlp(ss2, p['mlp_w1'], b1[i], p['mlp_w2'], b2[i])
        parts = _msg_fwd_sc(W, h1, row_m, col_m, z128)
        t, h = _node_t(parts, p['lin2_w'], l2b[i], p['lin_w'], lnb[i])
        Ws.append(W)
        h1s.append(h1)
        ts.append(t)
        if i + 1 < NINT:
            h1 = _mm(h, inter[i + 1]['lin1_w'])

    en64, st64 = _readout(h, seg2, params['energy_w'],
                          params['energy_b'].reshape(1, 1),
                          params['stress_w'],
                          params['stress_b'].reshape(1, 6))
    energy = en64.reshape(NSEG)
    stress = st64

    g_h = jnp.broadcast_to(params['energy_w'][:, 0][None, :],
                           (N_NODES, HIDDEN))
    gacc = jnp.zeros((N_EDGES, 1), F32)
    for i in reversed(range(NINT)):
        p = inter[i]
        g_agg = _node_bwd(g_h, ts[i], p['lin_w'], p['lin2_w'])
        gW, gh1_parts = _msg_bwd_sc(Ws[i], h1s[i], g_agg, row_m, col_m,
                                    z128)
        gacc = _edge_bwd(ss2, gW, p['mlp_w1'], b1[i], p['mlp_w2'], b2[i],
                         gacc, final=(i == 0))
        if i > 0:
            g_h = _node_bwd_h1(gh1_parts, p['lin1_w'])

    gs_f = gacc.reshape(NW, CF, KF)
    fparts = _force_sc(px, py, pz, row_f, col_f, gs_f, z16)
    forces = _force_fin(fparts)
    return forces, energy, stress


# speedup vs baseline: 1.9316x; 1.9316x over previous
import functools
import math

import jax
import jax.numpy as jnp
from jax import lax
from jax.experimental import pallas as pl
from jax.experimental.pallas import tpu as pltpu
from jax.experimental.pallas import tpu_sc as plsc

F32 = jnp.float32
I32 = jnp.int32
HIDDEN = 128
NGAUSS = 50
NINT = 3
CUTOFF = 5.0
NSEG = 64
N_NODES = 10000
N_EDGES = 320000
SHIFT = math.log(2.0)
GAMMA = float(0.5 / (CUTOFF / (NGAUSS - 1)) ** 2)
PI_C = float(math.pi / CUTOFF)

NC = 2
NS = 16
NW = NC * NS
EW = N_EDGES // NW
KM = 80
CM = EW // KM
NPAD = 10240
NPT = NPAD // NS

BE = 1280
GE = N_EDGES // BE
BN = 1000
GN = N_NODES // BN

def _mesh():
    return plsc.VectorSubcoreMesh(core_axis_name="c", subcore_axis_name="s")


def _ssp(x):
    return jnp.maximum(x, 0.0) + jnp.log1p(jnp.exp(-jnp.abs(x))) - SHIFT


def _sig(x):
    return jax.nn.sigmoid(x)


def _dotT(a, b):
    return lax.dot_general(a, b, (((1,), (1,)), ((), ())),
                           preferred_element_type=F32)


def _dot(a, b):
    return jnp.dot(a, b, preferred_element_type=F32)


def _offsets_row():
    return (lax.broadcasted_iota(I32, (1, NGAUSS), 1).astype(F32)
            * (CUTOFF / (NGAUSS - 1)))


def _node0_body(an_ref, emb_ref, lin1_ref, out_ref):
    an = an_ref[...]
    ids = lax.broadcasted_iota(I32, (BN, 100), 1)
    oh = (an == ids).astype(F32)
    h0 = _dot(oh, emb_ref[...])
    out_ref[...] = _dot(h0, lin1_ref[...])


def _node0(an2, emb, lin1):
    return pl.pallas_call(
        _node0_body,
        grid=(GN,),
        in_specs=[
            pl.BlockSpec((BN, 1), lambda i: (i, 0)),
            pl.BlockSpec((100, HIDDEN), lambda i: (0, 0)),
            pl.BlockSpec((HIDDEN, HIDDEN), lambda i: (0, 0)),
        ],
        out_specs=pl.BlockSpec((BN, HIDDEN), lambda i: (i, 0)),
        out_shape=jax.ShapeDtypeStruct((N_NODES, HIDDEN), F32),
    )(an2, emb, lin1)


def _edge_mlp_body(ss_ref, w1_ref, b1_ref, w2_ref, b2_ref, out_ref):
    ew = jnp.sqrt(ss_ref[...] + 1e-12)
    off = _offsets_row()
    ea = jnp.exp(-GAMMA * (ew - off) ** 2)
    a1 = _dot(ea, w1_ref[...]) + b1_ref[...]
    sp = _ssp(a1)
    filt = _dot(sp, w2_ref[...]) + b2_ref[...]
    cth = 0.5 * (jnp.cos(ew * PI_C) + 1.0)
    out_ref[...] = cth * filt


def _edge_mlp(ss2, w1, b1, w2, b2):
    return pl.pallas_call(
        _edge_mlp_body,
        grid=(GE,),
        in_specs=[
            pl.BlockSpec((BE, 1), lambda i: (i, 0)),
            pl.BlockSpec((NGAUSS, HIDDEN), lambda i: (0, 0)),
            pl.BlockSpec((1, HIDDEN), lambda i: (0, 0)),
            pl.BlockSpec((HIDDEN, HIDDEN), lambda i: (0, 0)),
            pl.BlockSpec((1, HIDDEN), lambda i: (0, 0)),
        ],
        out_specs=pl.BlockSpec((BE, HIDDEN), lambda i: (i, 0)),
        out_shape=jax.ShapeDtypeStruct((N_EDGES, HIDDEN), F32),
    )(ss2, w1, b1, w2, b2)


def _node_t_body(parts_ref, lin2_ref, b2_ref, linw_ref, linb_ref,
                 t_ref, h_ref):
    agg = parts_ref[0] + parts_ref[1]
    t = _dot(agg, lin2_ref[...]) + b2_ref[...]
    t_ref[...] = t
    h_ref[...] = _dot(_ssp(t), linw_ref[...]) + linb_ref[...]


def _node_t(parts, lin2, b2, linw, linb):
    return pl.pallas_call(
        _node_t_body,
        grid=(GN,),
        in_specs=[
            pl.BlockSpec((2, BN, HIDDEN), lambda i: (0, i, 0)),
            pl.BlockSpec((HIDDEN, HIDDEN), lambda i: (0, 0)),
            pl.BlockSpec((1, HIDDEN), lambda i: (0, 0)),
            pl.BlockSpec((HIDDEN, HIDDEN), lambda i: (0, 0)),
            pl.BlockSpec((1, HIDDEN), lambda i: (0, 0)),
        ],
        out_specs=[
            pl.BlockSpec((BN, HIDDEN), lambda i: (i, 0)),
            pl.BlockSpec((BN, HIDDEN), lambda i: (i, 0)),
        ],
        out_shape=[
            jax.ShapeDtypeStruct((N_NODES, HIDDEN), F32),
            jax.ShapeDtypeStruct((N_NODES, HIDDEN), F32),
        ],
    )(parts, lin2, b2, linw, linb)


def _mm_body(x_ref, w_ref, o_ref):
    o_ref[...] = _dot(x_ref[...], w_ref[...])


def _mm(x, w):
    return pl.pallas_call(
        _mm_body,
        grid=(GN,),
        in_specs=[
            pl.BlockSpec((BN, HIDDEN), lambda i: (i, 0)),
            pl.BlockSpec((HIDDEN, HIDDEN), lambda i: (0, 0)),
        ],
        out_specs=pl.BlockSpec((BN, HIDDEN), lambda i: (i, 0)),
        out_shape=jax.ShapeDtypeStruct((N_NODES, HIDDEN), F32),
    )(x, w)


def _node_bwd_body(g_ref, t_ref, linw_ref, lin2_ref, o_ref):
    gu = _dotT(g_ref[...], linw_ref[...])
    gt = gu * _sig(t_ref[...])
    o_ref[...] = _dotT(gt, lin2_ref[...])


def _node_bwd(g, t, linw, lin2):
    return pl.pallas_call(
        _node_bwd_body,
        grid=(GN,),
        in_specs=[
            pl.BlockSpec((BN, HIDDEN), lambda i: (i, 0)),
            pl.BlockSpec((BN, HIDDEN), lambda i: (i, 0)),
            pl.BlockSpec((HIDDEN, HIDDEN), lambda i: (0, 0)),
            pl.BlockSpec((HIDDEN, HIDDEN), lambda i: (0, 0)),
        ],
        out_specs=pl.BlockSpec((BN, HIDDEN), lambda i: (i, 0)),
        out_shape=jax.ShapeDtypeStruct((N_NODES, HIDDEN), F32),
    )(g, t, linw, lin2)


def _node_bwd_h1_body(parts_ref, lin1_ref, o_ref):
    s = parts_ref[0] + parts_ref[1]
    o_ref[...] = _dotT(s, lin1_ref[...])


def _node_bwd_h1(parts, lin1):
    return pl.pallas_call(
        _node_bwd_h1_body,
        grid=(GN,),
        in_specs=[
            pl.BlockSpec((2, BN, HIDDEN), lambda i: (0, i, 0)),
            pl.BlockSpec((HIDDEN, HIDDEN), lambda i: (0, 0)),
        ],
        out_specs=pl.BlockSpec((BN, HIDDEN), lambda i: (i, 0)),
        out_shape=jax.ShapeDtypeStruct((N_NODES, HIDDEN), F32),
    )(parts, lin1)


def _edge_bwd_body(ss_ref, gW_ref, w1_ref, b1_ref, w2_ref, b2_ref, acc_ref,
                   out_ref, *, final):
    ew = jnp.sqrt(ss_ref[...] + 1e-12)
    off = _offsets_row()
    ea = jnp.exp(-GAMMA * (ew - off) ** 2)
    a1 = _dot(ea, w1_ref[...]) + b1_ref[...]
    sp = _ssp(a1)
    filt = _dot(sp, w2_ref[...]) + b2_ref[...]
    cth = 0.5 * (jnp.cos(ew * PI_C) + 1.0)
    gW = gW_ref[...]
    g_c = jnp.sum(gW * filt, axis=1, keepdims=True)
    g_filt = cth * gW
    g_sp = _dotT(g_filt, w2_ref[...])
    g_a1 = g_sp * _sig(a1)
    g_ea = _dotT(g_a1, w1_ref[...])
    dea = ea * (-2.0 * GAMMA) * (ew - off)
    contrib = (jnp.sum(g_ea * dea, axis=1, keepdims=True)
               + g_c * (-0.5 * PI_C) * jnp.sin(ew * PI_C))
    tot = acc_ref[...] + contrib
    if final:
        tot = tot / ew
    out_ref[...] = tot


def _edge_bwd(ss2, gW, w1, b1, w2, b2, acc, final):
    return pl.pallas_call(
        functools.partial(_edge_bwd_body, final=final),
        grid=(GE,),
        in_specs=[
            pl.BlockSpec((BE, 1), lambda i: (i, 0)),
            pl.BlockSpec((BE, HIDDEN), lambda i: (i, 0)),
            pl.BlockSpec((NGAUSS, HIDDEN), lambda i: (0, 0)),
            pl.BlockSpec((1, HIDDEN), lambda i: (0, 0)),
            pl.BlockSpec((HIDDEN, HIDDEN), lambda i: (0, 0)),
            pl.BlockSpec((1, HIDDEN), lambda i: (0, 0)),
            pl.BlockSpec((BE, 1), lambda i: (i, 0)),
        ],
        out_specs=pl.BlockSpec((BE, 1), lambda i: (i, 0)),
        out_shape=jax.ShapeDtypeStruct((N_EDGES, 1), F32),
    )(ss2, gW, w1, b1, w2, b2, acc)


def _readout_body(h_ref, seg_ref, ew_ref, eb_ref, sw_ref, sb_ref,
                  en_ref, st_ref):
    i = pl.program_id(0)
    h = h_ref[...]
    e = _dot(h, ew_ref[...]) + eb_ref[...]
    hs = _dot(h, sw_ref[...]) + sb_ref[...]
    ids = lax.broadcasted_iota(I32, (BN, NSEG), 1)
    oh = (seg_ref[...] == ids).astype(F32)
    en_c = lax.dot_general(oh, e, (((0,), (0,)), ((), ())),
                           preferred_element_type=F32)
    st_c = lax.dot_general(oh, hs, (((0,), (0,)), ((), ())),
                           preferred_element_type=F32)

    @pl.when(i == 0)
    def _():
        en_ref[...] = en_c
        st_ref[...] = st_c

    @pl.when(i != 0)
    def _():
        en_ref[...] += en_c
        st_ref[...] += st_c


def _readout(h, seg2, energy_w, energy_b, stress_w, stress_b):
    return pl.pallas_call(
        _readout_body,
        grid=(GN,),
        in_specs=[
            pl.BlockSpec((BN, HIDDEN), lambda i: (i, 0)),
            pl.BlockSpec((BN, 1), lambda i: (i, 0)),
            pl.BlockSpec((HIDDEN, 1), lambda i: (0, 0)),
            pl.BlockSpec((1, 1), lambda i: (0, 0)),
            pl.BlockSpec((HIDDEN, 6), lambda i: (0, 0)),
            pl.BlockSpec((1, 6), lambda i: (0, 0)),
        ],
        out_specs=[
            pl.BlockSpec((NSEG, 1), lambda i: (0, 0)),
            pl.BlockSpec((NSEG, 6), lambda i: (0, 0)),
        ],
        out_shape=[
            jax.ShapeDtypeStruct((NSEG, 1), F32),
            jax.ShapeDtypeStruct((NSEG, 6), F32),
        ],
    )(h, seg2, energy_w, energy_b, stress_w, stress_b)


def _force_fin_body(parts_ref, o_ref):
    s = parts_ref[0] + parts_ref[1]
    o_ref[...] = -s[:, :3]


def _force_fin(parts):
    return pl.pallas_call(
        _force_fin_body,
        grid=(GN,),
        in_specs=[pl.BlockSpec((2, BN, HIDDEN), lambda i: (0, i, 0))],
        out_specs=pl.BlockSpec((BN, 3), lambda i: (i, 0)),
        out_shape=jax.ShapeDtypeStruct((N_NODES, 3), F32),
    )(parts)


def _geom_sc_body(px_hbm, py_hbm, pz_hbm, row_hbm, col_hbm,
                  dx_hbm, dy_hbm, dz_hbm, ss_hbm,
                  pxb, pyb, pzb, rbuf, cbuf, dxb, dyb, dzb, ssb):
    c = lax.axis_index("c")
    s = lax.axis_index("s")
    wid = c * NS + s
    pltpu.sync_copy(px_hbm, pxb)
    pltpu.sync_copy(py_hbm, pyb)
    pltpu.sync_copy(pz_hbm, pzb)
    base = wid * EW
    pltpu.sync_copy(row_hbm.at[pl.ds(base, EW)], rbuf)
    pltpu.sync_copy(col_hbm.at[pl.ds(base, EW)], cbuf)

    def grp(g, carry):
        sl = pl.ds(g * 16, 16)
        ri = rbuf[sl]
        cj = cbuf[sl]
        ax = plsc.load_gather(pxb, [ri]) - plsc.load_gather(pxb, [cj])
        ay = plsc.load_gather(pyb, [ri]) - plsc.load_gather(pyb, [cj])
        az = plsc.load_gather(pzb, [ri]) - plsc.load_gather(pzb, [cj])
        dxb[sl] = ax
        dyb[sl] = ay
        dzb[sl] = az
        ssb[sl] = ax * ax + ay * ay + az * az
        return carry

    lax.fori_loop(0, EW // 16, grp, 0)
    pltpu.sync_copy(dxb, dx_hbm.at[pl.ds(base, EW)])
    pltpu.sync_copy(dyb, dy_hbm.at[pl.ds(base, EW)])
    pltpu.sync_copy(dzb, dz_hbm.at[pl.ds(base, EW)])
    pltpu.sync_copy(ssb, ss_hbm.at[pl.ds(base, EW)])


@functools.lru_cache(maxsize=None)
def _build_geom_sc():
    return pl.kernel(
        _geom_sc_body,
        out_type=(
            jax.ShapeDtypeStruct((N_EDGES,), F32),
            jax.ShapeDtypeStruct((N_EDGES,), F32),
            jax.ShapeDtypeStruct((N_EDGES,), F32),
            jax.ShapeDtypeStruct((N_EDGES,), F32),
        ),
        mesh=_mesh(),
        scratch_types=[
            pltpu.VMEM((N_NODES,), F32),
            pltpu.VMEM((N_NODES,), F32),
            pltpu.VMEM((N_NODES,), F32),
            pltpu.VMEM((EW,), I32),
            pltpu.VMEM((EW,), I32),
            pltpu.VMEM((EW,), F32),
            pltpu.VMEM((EW,), F32),
            pltpu.VMEM((EW,), F32),
            pltpu.VMEM((EW,), F32),
        ],
        compiler_params=pltpu.CompilerParams(needs_layout_passes=False),
    )


def _geom_sc(px, py, pz, row, col):
    return _build_geom_sc()(px, py, pz, row, col)


def _msg_fwd_sc_body(W_hbm, h1_hbm, row_hbm, col_hbm, z_hbm, out_hbm,
                     rbuf, cbuf, wbuf, hbuf, agg_sp, sem):
    c = lax.axis_index("c")
    s = lax.axis_index("s")
    wid = c * NS + s
    pltpu.sync_copy(z_hbm, agg_sp.at[pl.ds(s * NPT, NPT)])
    plsc.subcore_barrier()
    base = wid * EW

    def chunk(j, carry):
        e0 = base + j * KM
        pltpu.sync_copy(col_hbm.at[pl.ds(e0, KM)], cbuf)
        d = pltpu.async_copy(h1_hbm.at[cbuf], hbuf, sem)
        pltpu.sync_copy(row_hbm.at[pl.ds(e0, KM)], rbuf)
        pltpu.sync_copy(W_hbm.at[pl.ds(e0, KM)], wbuf)
        d.wait()

        def rowloop(r, carry2):
            for f in range(HIDDEN // 16):
                sl = pl.ds(f * 16, 16)
                hbuf[r, sl] = wbuf[r, sl] * hbuf[r, sl]
            return carry2

        lax.fori_loop(0, KM, rowloop, 0)
        pltpu.sync_copy(hbuf, agg_sp.at[rbuf], add=True)
        return carry

    lax.fori_loop(0, CM, chunk, 0)
    plsc.subcore_barrier()
    pltpu.sync_copy(agg_sp.at[pl.ds(s * NPT, NPT)],
                    out_hbm.at[c, pl.ds(s * NPT, NPT)])


@functools.lru_cache(maxsize=None)
def _build_msg_fwd_sc():
    return pl.kernel(
        _msg_fwd_sc_body,
        out_type=jax.ShapeDtypeStruct((NC, NPAD, HIDDEN), F32),
        mesh=_mesh(),
        scratch_types=[
            pltpu.VMEM((KM,), I32),
            pltpu.VMEM((KM,), I32),
            pltpu.VMEM((KM, HIDDEN), F32),
            pltpu.VMEM((KM, HIDDEN), F32),
            pltpu.VMEM_SHARED((NPAD, HIDDEN), F32),
            pltpu.SemaphoreType.DMA,
        ],
        compiler_params=pltpu.CompilerParams(needs_layout_passes=False),
    )


def _msg_fwd_sc(W, h1, row, col, z):
    return _build_msg_fwd_sc()(W, h1, row, col, z)


def _msg_bwd_sc_body(W_hbm, h1_hbm, gagg_hbm, row_hbm, col_hbm, z_hbm,
                     gw_hbm, out_hbm, rbuf, cbuf, wbuf, hbuf, gbuf,
                     gh1_sp, sem):
    c = lax.axis_index("c")
    s = lax.axis_index("s")
    wid = c * NS + s
    pltpu.sync_copy(z_hbm, gh1_sp.at[pl.ds(s * NPT, NPT)])
    plsc.subcore_barrier()
    base = wid * EW

    def chunk(j, carry):
        e0 = base + j * KM
        pltpu.sync_copy(row_hbm.at[pl.ds(e0, KM)], rbuf)
        d1 = pltpu.async_copy(gagg_hbm.at[rbuf], gbuf, sem)
        pltpu.sync_copy(col_hbm.at[pl.ds(e0, KM)], cbuf)
        d2 = pltpu.async_copy(h1_hbm.at[cbuf], hbuf, sem)
        pltpu.sync_copy(W_hbm.at[pl.ds(e0, KM)], wbuf)
        d1.wait()
        d2.wait()

        def rowloop(r, carry2):
            for f in range(HIDDEN // 16):
                sl = pl.ds(f * 16, 16)
                g16 = gbuf[r, sl]
                wbuf[r, sl] = g16 * wbuf[r, sl]
                gbuf[r, sl] = g16 * hbuf[r, sl]
            return carry2

        lax.fori_loop(0, KM, rowloop, 0)
        pltpu.sync_copy(wbuf, gh1_sp.at[cbuf], add=True)
        pltpu.sync_copy(gbuf, gw_hbm.at[pl.ds(e0, KM)])
        return carry

    lax.fori_loop(0, CM, chunk, 0)
    plsc.subcore_barrier()
    pltpu.sync_copy(gh1_sp.at[pl.ds(s * NPT, NPT)],
                    out_hbm.at[c, pl.ds(s * NPT, NPT)])


@functools.lru_cache(maxsize=None)
def _build_msg_bwd_sc():
    return pl.kernel(
        _msg_bwd_sc_body,
        out_type=(
            jax.ShapeDtypeStruct((N_EDGES, HIDDEN), F32),
            jax.ShapeDtypeStruct((NC, NPAD, HIDDEN), F32),
        ),
        mesh=_mesh(),
        scratch_types=[
            pltpu.VMEM((KM,), I32),
            pltpu.VMEM((KM,), I32),
            pltpu.VMEM((KM, HIDDEN), F32),
            pltpu.VMEM((KM, HIDDEN), F32),
            pltpu.VMEM((KM, HIDDEN), F32),
            pltpu.VMEM_SHARED((NPAD, HIDDEN), F32),
            pltpu.SemaphoreType.DMA,
        ],
        compiler_params=pltpu.CompilerParams(needs_layout_passes=False),
    )


def _msg_bwd_sc(W, h1, gagg, row, col, z):
    return _build_msg_bwd_sc()(W, h1, gagg, row, col, z)


def _force_sc_body(dx_hbm, dy_hbm, dz_hbm, gs_hbm, row_hbm, col_hbm,
                   out_hbm, rbuf, cbuf, dxb, dyb, dzb, gsb, sbuf, nbuf,
                   fsp):
    c = lax.axis_index("c")
    s = lax.axis_index("s")
    wid = c * NS + s
    base = wid * EW

    zero16 = jnp.zeros((16,), F32)

    def zrow(r, carry):
        for f in range(HIDDEN // 16):
            sl = pl.ds(f * 16, 16)
            sbuf[r, sl] = zero16
            nbuf[r, sl] = zero16
        return carry

    lax.fori_loop(0, KM, zrow, 0)

    def ztab(q, carry):
        pltpu.sync_copy(sbuf, fsp.at[pl.ds(s * NPT + q * KM, KM)])
        return carry

    lax.fori_loop(0, NPT // KM, ztab, 0)
    plsc.subcore_barrier()

    lanes = lax.iota(I32, 16)
    col0 = jnp.zeros((16,), I32)
    col1 = col0 + 1
    col2 = col0 + 2

    def chunk(ci, carry):
        e0 = base + ci * KM
        pltpu.sync_copy(row_hbm.at[pl.ds(e0, KM)], rbuf)
        pltpu.sync_copy(col_hbm.at[pl.ds(e0, KM)], cbuf)
        pltpu.sync_copy(dx_hbm.at[pl.ds(e0, KM)], dxb)
        pltpu.sync_copy(dy_hbm.at[pl.ds(e0, KM)], dyb)
        pltpu.sync_copy(dz_hbm.at[pl.ds(e0, KM)], dzb)
        pltpu.sync_copy(gs_hbm.at[pl.ds(e0, KM)], gsb)

        def grp(g, carry2):
            sl = pl.ds(g * 16, 16)
            gsv = gsb[sl]
            vx = dxb[sl] * gsv
            vy = dyb[sl] * gsv
            vz = dzb[sl] * gsv
            rows = lanes + g * 16
            plsc.store_scatter(sbuf, [rows, col0], vx)
            plsc.store_scatter(sbuf, [rows, col1], vy)
            plsc.store_scatter(sbuf, [rows, col2], vz)
            plsc.store_scatter(nbuf, [rows, col0], -vx)
            plsc.store_scatter(nbuf, [rows, col1], -vy)
            plsc.store_scatter(nbuf, [rows, col2], -vz)
            return carry2

        lax.fori_loop(0, KM // 16, grp, 0)
        pltpu.sync_copy(sbuf, fsp.at[rbuf], add=True)
        pltpu.sync_copy(nbuf, fsp.at[cbuf], add=True)
        return carry

    lax.fori_loop(0, CM, chunk, 0)
    plsc.subcore_barrier()
    pltpu.sync_copy(fsp.at[pl.ds(s * NPT, NPT)],
                    out_hbm.at[c, pl.ds(s * NPT, NPT)])


@functools.lru_cache(maxsize=None)
def _build_force_sc():
    return pl.kernel(
        _force_sc_body,
        out_type=jax.ShapeDtypeStruct((NC, NPAD, HIDDEN), F32),
        mesh=_mesh(),
        scratch_types=[
            pltpu.VMEM((KM,), I32),
            pltpu.VMEM((KM,), I32),
            pltpu.VMEM((KM,), F32),
            pltpu.VMEM((KM,), F32),
            pltpu.VMEM((KM,), F32),
            pltpu.VMEM((KM,), F32),
            pltpu.VMEM((KM, HIDDEN), F32),
            pltpu.VMEM((KM, HIDDEN), F32),
            pltpu.VMEM_SHARED((NPAD, HIDDEN), F32),
        ],
        compiler_params=pltpu.CompilerParams(needs_layout_passes=False),
    )


def _force_sc(dx, dy, dz, gs, row, col):
    return _build_force_sc()(dx, dy, dz, gs, row, col)


def kernel(atomic_numbers, positions, edge_index, structure_index, params):
    inter = params['interactions']
    row = edge_index[0].astype(I32)
    col = edge_index[1].astype(I32)
    px = positions[:, 0]
    py = positions[:, 1]
    pz = positions[:, 2]
    an2 = atomic_numbers.astype(I32).reshape(N_NODES, 1)
    seg2 = structure_index.astype(I32).reshape(N_NODES, 1)
    z128 = jnp.zeros((NPT, HIDDEN), F32)

    b1 = [p['mlp_b1'].reshape(1, HIDDEN) for p in inter]
    b2 = [p['mlp_b2'].reshape(1, HIDDEN) for p in inter]
    l2b = [p['lin2_b'].reshape(1, HIDDEN) for p in inter]
    lnb = [p['lin_b'].reshape(1, HIDDEN) for p in inter]

    dx, dy, dz, sumsq = _geom_sc(px, py, pz, row, col)
    ss2 = sumsq.reshape(N_EDGES, 1)

    h1 = _node0(an2, params['emb'], inter[0]['lin1_w'])
    Ws = []
    h1s = []
    ts = []
    h = None
    for i in range(NINT):
        p = inter[i]
        W = _edge_mlp(ss2, p['mlp_w1'], b1[i], p['mlp_w2'], b2[i])
        parts = _msg_fwd_sc(W, h1, row, col, z128)
        t, h = _node_t(parts, p['lin2_w'], l2b[i], p['lin_w'], lnb[i])
        Ws.append(W)
        h1s.append(h1)
        ts.append(t)
        if i + 1 < NINT:
            h1 = _mm(h, inter[i + 1]['lin1_w'])

    en64, st64 = _readout(h, seg2, params['energy_w'],
                          params['energy_b'].reshape(1, 1),
                          params['stress_w'],
                          params['stress_b'].reshape(1, 6))
    energy = en64.reshape(NSEG)
    stress = st64

    g_h = jnp.broadcast_to(params['energy_w'][:, 0][None, :],
                           (N_NODES, HIDDEN))
    gacc = jnp.zeros((N_EDGES, 1), F32)
    for i in reversed(range(NINT)):
        p = inter[i]
        g_agg = _node_bwd(g_h, ts[i], p['lin_w'], p['lin2_w'])
        gW, gh1_parts = _msg_bwd_sc(Ws[i], h1s[i], g_agg, row, col, z128)
        gacc = _edge_bwd(ss2, gW, p['mlp_w1'], b1[i], p['mlp_w2'], b2[i],
                         gacc, final=(i == 0))
        if i > 0:
            g_h = _node_bwd_h1(gh1_parts, p['lin1_w'])

    gs1 = gacc.reshape(N_EDGES)
    fparts = _force_sc(dx, dy, dz, gs1, row, col)
    forces = _force_fin(fparts)
    return forces, energy, stress


# speedup vs baseline: 2.0924x; 1.0833x over previous
import functools
import math

import jax
import jax.numpy as jnp
from jax import lax
from jax.experimental import pallas as pl
from jax.experimental.pallas import tpu as pltpu
from jax.experimental.pallas import tpu_sc as plsc

F32 = jnp.float32
I32 = jnp.int32
HIDDEN = 128
NGAUSS = 50
NINT = 3
CUTOFF = 5.0
NSEG = 64
N_NODES = 10000
N_EDGES = 320000
SHIFT = math.log(2.0)
GAMMA = float(0.5 / (CUTOFF / (NGAUSS - 1)) ** 2)
PI_C = float(math.pi / CUTOFF)

NC = 2
NS = 16
NW = NC * NS
EW = N_EDGES // NW
KM = 80
CM = EW // KM
NPAD = 10240
NPT = NPAD // NS
KB = 40
CB = EW // KB

BE = 1280
GE = N_EDGES // BE
BN = 1000
GN = N_NODES // BN

def _mesh():
    return plsc.VectorSubcoreMesh(core_axis_name="c", subcore_axis_name="s")


def _ssp(x):
    return jax.nn.softplus(x) - SHIFT


def _sig(x):
    return jax.nn.sigmoid(x)


def _dotT(a, b):
    return lax.dot_general(a, b, (((1,), (1,)), ((), ())),
                           preferred_element_type=F32)


def _dot(a, b):
    return jnp.dot(a, b, preferred_element_type=F32)


def _offsets_row():
    return (lax.broadcasted_iota(I32, (1, NGAUSS), 1).astype(F32)
            * (CUTOFF / (NGAUSS - 1)))


def _node0_body(an_ref, emb_ref, lin1_ref, out_ref):
    an = an_ref[...]
    ids = lax.broadcasted_iota(I32, (BN, 100), 1)
    oh = (an == ids).astype(F32)
    h0 = _dot(oh, emb_ref[...])
    out_ref[...] = _dot(h0, lin1_ref[...])


def _node0(an2, emb, lin1):
    return pl.pallas_call(
        _node0_body,
        grid=(GN,),
        in_specs=[
            pl.BlockSpec((BN, 1), lambda i: (i, 0)),
            pl.BlockSpec((100, HIDDEN), lambda i: (0, 0)),
            pl.BlockSpec((HIDDEN, HIDDEN), lambda i: (0, 0)),
        ],
        out_specs=pl.BlockSpec((BN, HIDDEN), lambda i: (i, 0)),
        out_shape=jax.ShapeDtypeStruct((N_NODES, HIDDEN), F32),
    )(an2, emb, lin1)


def _edge_mlp_body(ss_ref, off_ref, ng_ref, w1_ref, b1_ref, w2_ref,
                   b2_ref, out_ref):
    ew = jnp.sqrt(ss_ref[...] + 1e-12)
    off = off_ref[...]
    ng = ng_ref[...]
    ea = jnp.exp(ng * (ew - off) ** 2)
    a1 = _dot(ea, w1_ref[...]) + b1_ref[...]
    sp = _ssp(a1)
    filt = _dot(sp, w2_ref[...]) + b2_ref[...]
    cth = 0.5 * (jnp.cos(ew * math.pi / CUTOFF) + 1.0)
    out_ref[...] = cth * filt


def _edge_mlp(ss2, off, ng, w1, b1, w2, b2):
    return pl.pallas_call(
        _edge_mlp_body,
        grid=(GE,),
        in_specs=[
            pl.BlockSpec((BE, 1), lambda i: (i, 0)),
            pl.BlockSpec((1, NGAUSS), lambda i: (0, 0)),
            pl.BlockSpec((1, 1), lambda i: (0, 0)),
            pl.BlockSpec((NGAUSS, HIDDEN), lambda i: (0, 0)),
            pl.BlockSpec((1, HIDDEN), lambda i: (0, 0)),
            pl.BlockSpec((HIDDEN, HIDDEN), lambda i: (0, 0)),
            pl.BlockSpec((1, HIDDEN), lambda i: (0, 0)),
        ],
        out_specs=pl.BlockSpec((BE, HIDDEN), lambda i: (i, 0)),
        out_shape=jax.ShapeDtypeStruct((N_EDGES, HIDDEN), F32),
    )(ss2, off, ng, w1, b1, w2, b2)


def _node_t_body(parts_ref, lin2_ref, b2_ref, linw_ref, linb_ref,
                 t_ref, h_ref):
    agg = parts_ref[0] + parts_ref[1]
    t = _dot(agg, lin2_ref[...]) + b2_ref[...]
    t_ref[...] = t
    h_ref[...] = _dot(_ssp(t), linw_ref[...]) + linb_ref[...]


def _node_t(parts, lin2, b2, linw, linb):
    return pl.pallas_call(
        _node_t_body,
        grid=(GN,),
        in_specs=[
            pl.BlockSpec((2, BN, HIDDEN), lambda i: (0, i, 0)),
            pl.BlockSpec((HIDDEN, HIDDEN), lambda i: (0, 0)),
            pl.BlockSpec((1, HIDDEN), lambda i: (0, 0)),
            pl.BlockSpec((HIDDEN, HIDDEN), lambda i: (0, 0)),
            pl.BlockSpec((1, HIDDEN), lambda i: (0, 0)),
        ],
        out_specs=[
            pl.BlockSpec((BN, HIDDEN), lambda i: (i, 0)),
            pl.BlockSpec((BN, HIDDEN), lambda i: (i, 0)),
        ],
        out_shape=[
            jax.ShapeDtypeStruct((N_NODES, HIDDEN), F32),
            jax.ShapeDtypeStruct((N_NODES, HIDDEN), F32),
        ],
    )(parts, lin2, b2, linw, linb)


def _mm_body(x_ref, w_ref, o_ref):
    o_ref[...] = _dot(x_ref[...], w_ref[...])


def _mm(x, w):
    return pl.pallas_call(
        _mm_body,
        grid=(GN,),
        in_specs=[
            pl.BlockSpec((BN, HIDDEN), lambda i: (i, 0)),
            pl.BlockSpec((HIDDEN, HIDDEN), lambda i: (0, 0)),
        ],
        out_specs=pl.BlockSpec((BN, HIDDEN), lambda i: (i, 0)),
        out_shape=jax.ShapeDtypeStruct((N_NODES, HIDDEN), F32),
    )(x, w)


def _node_bwd_body(g_ref, t_ref, linw_ref, lin2_ref, o_ref):
    gu = _dotT(g_ref[...], linw_ref[...])
    gt = gu * _sig(t_ref[...])
    o_ref[...] = _dotT(gt, lin2_ref[...])


def _node_bwd(g, t, linw, lin2):
    return pl.pallas_call(
        _node_bwd_body,
        grid=(GN,),
        in_specs=[
            pl.BlockSpec((BN, HIDDEN), lambda i: (i, 0)),
            pl.BlockSpec((BN, HIDDEN), lambda i: (i, 0)),
            pl.BlockSpec((HIDDEN, HIDDEN), lambda i: (0, 0)),
            pl.BlockSpec((HIDDEN, HIDDEN), lambda i: (0, 0)),
        ],
        out_specs=pl.BlockSpec((BN, HIDDEN), lambda i: (i, 0)),
        out_shape=jax.ShapeDtypeStruct((N_NODES, HIDDEN), F32),
    )(g, t, linw, lin2)


def _node_bwd_h1_body(parts_ref, lin1_ref, o_ref):
    s = parts_ref[0] + parts_ref[1]
    o_ref[...] = _dotT(s, lin1_ref[...])


def _node_bwd_h1(parts, lin1):
    return pl.pallas_call(
        _node_bwd_h1_body,
        grid=(GN,),
        in_specs=[
            pl.BlockSpec((2, BN, HIDDEN), lambda i: (0, i, 0)),
            pl.BlockSpec((HIDDEN, HIDDEN), lambda i: (0, 0)),
        ],
        out_specs=pl.BlockSpec((BN, HIDDEN), lambda i: (i, 0)),
        out_shape=jax.ShapeDtypeStruct((N_NODES, HIDDEN), F32),
    )(parts, lin1)


def _edge_bwd_body(ss_ref, off_ref, ng_ref, gW_ref, w1_ref, b1_ref,
                   w2_ref, b2_ref, acc_ref, out_ref, *, final):
    ew = jnp.sqrt(ss_ref[...] + 1e-12)
    off = off_ref[...]
    ng = ng_ref[...]
    y = ew - off
    ea = jnp.exp(ng * y ** 2)
    a1 = _dot(ea, w1_ref[...]) + b1_ref[...]
    sp = _ssp(a1)
    filt = _dot(sp, w2_ref[...]) + b2_ref[...]
    u = ew * math.pi / CUTOFF
    gW = gW_ref[...]
    g_c = jnp.sum(gW * filt, axis=1, keepdims=True)
    cth = 0.5 * (jnp.cos(u) + 1.0)
    g_filt = cth * gW
    g_sp = _dotT(g_filt, w2_ref[...])
    g_a1 = g_sp * _sig(a1)
    g_ea = _dotT(g_a1, w1_ref[...])
    dterm = ((g_ea * ea) * ng) * (2.0 * y)
    cterm = (((g_c * 0.5) * -jnp.sin(u)) / CUTOFF) * math.pi
    contrib = jnp.sum(dterm, axis=1, keepdims=True) + cterm
    tot = acc_ref[...] + contrib
    if final:
        tot = tot / ew
    out_ref[...] = tot


def _edge_bwd(ss2, off, ng, gW, w1, b1, w2, b2, acc, final):
    return pl.pallas_call(
        functools.partial(_edge_bwd_body, final=final),
        grid=(GE,),
        in_specs=[
            pl.BlockSpec((BE, 1), lambda i: (i, 0)),
            pl.BlockSpec((1, NGAUSS), lambda i: (0, 0)),
            pl.BlockSpec((1, 1), lambda i: (0, 0)),
            pl.BlockSpec((BE, HIDDEN), lambda i: (i, 0)),
            pl.BlockSpec((NGAUSS, HIDDEN), lambda i: (0, 0)),
            pl.BlockSpec((1, HIDDEN), lambda i: (0, 0)),
            pl.BlockSpec((HIDDEN, HIDDEN), lambda i: (0, 0)),
            pl.BlockSpec((1, HIDDEN), lambda i: (0, 0)),
            pl.BlockSpec((BE, 1), lambda i: (i, 0)),
        ],
        out_specs=pl.BlockSpec((BE, 1), lambda i: (i, 0)),
        out_shape=jax.ShapeDtypeStruct((N_EDGES, 1), F32),
    )(ss2, off, ng, gW, w1, b1, w2, b2, acc)


def _readout_body(h_ref, seg_ref, ew_ref, eb_ref, sw_ref, sb_ref,
                  en_ref, st_ref):
    i = pl.program_id(0)
    h = h_ref[...]
    e = _dot(h, ew_ref[...]) + eb_ref[...]
    hs = _dot(h, sw_ref[...]) + sb_ref[...]
    ids = lax.broadcasted_iota(I32, (BN, NSEG), 1)
    oh = (seg_ref[...] == ids).astype(F32)
    en_c = lax.dot_general(oh, e, (((0,), (0,)), ((), ())),
                           preferred_element_type=F32)
    st_c = lax.dot_general(oh, hs, (((0,), (0,)), ((), ())),
                           preferred_element_type=F32)

    @pl.when(i == 0)
    def _():
        en_ref[...] = en_c
        st_ref[...] = st_c

    @pl.when(i != 0)
    def _():
        en_ref[...] += en_c
        st_ref[...] += st_c


def _readout(h, seg2, energy_w, energy_b, stress_w, stress_b):
    return pl.pallas_call(
        _readout_body,
        grid=(GN,),
        in_specs=[
            pl.BlockSpec((BN, HIDDEN), lambda i: (i, 0)),
            pl.BlockSpec((BN, 1), lambda i: (i, 0)),
            pl.BlockSpec((HIDDEN, 1), lambda i: (0, 0)),
            pl.BlockSpec((1, 1), lambda i: (0, 0)),
            pl.BlockSpec((HIDDEN, 6), lambda i: (0, 0)),
            pl.BlockSpec((1, 6), lambda i: (0, 0)),
        ],
        out_specs=[
            pl.BlockSpec((NSEG, 1), lambda i: (0, 0)),
            pl.BlockSpec((NSEG, 6), lambda i: (0, 0)),
        ],
        out_shape=[
            jax.ShapeDtypeStruct((NSEG, 1), F32),
            jax.ShapeDtypeStruct((NSEG, 6), F32),
        ],
    )(h, seg2, energy_w, energy_b, stress_w, stress_b)


def _force_fin_body(parts_ref, o_ref):
    s = parts_ref[0] + parts_ref[1]
    o_ref[...] = -s[:, :3]


def _force_fin(parts):
    return pl.pallas_call(
        _force_fin_body,
        grid=(GN,),
        in_specs=[pl.BlockSpec((2, BN, HIDDEN), lambda i: (0, i, 0))],
        out_specs=pl.BlockSpec((BN, 3), lambda i: (i, 0)),
        out_shape=jax.ShapeDtypeStruct((N_NODES, 3), F32),
    )(parts)


def _geom_sc_body(px_hbm, py_hbm, pz_hbm, row_hbm, col_hbm,
                  dx_hbm, dy_hbm, dz_hbm, ss_hbm,
                  pxb, pyb, pzb, rbuf, cbuf, dxb, dyb, dzb, ssb):
    c = lax.axis_index("c")
    s = lax.axis_index("s")
    wid = c * NS + s
    pltpu.sync_copy(px_hbm, pxb)
    pltpu.sync_copy(py_hbm, pyb)
    pltpu.sync_copy(pz_hbm, pzb)
    base = wid * EW
    pltpu.sync_copy(row_hbm.at[pl.ds(base, EW)], rbuf)
    pltpu.sync_copy(col_hbm.at[pl.ds(base, EW)], cbuf)

    def grp(g, carry):
        sl = pl.ds(g * 16, 16)
        ri = rbuf[sl]
        cj = cbuf[sl]
        ax = plsc.load_gather(pxb, [ri]) - plsc.load_gather(pxb, [cj])
        ay = plsc.load_gather(pyb, [ri]) - plsc.load_gather(pyb, [cj])
        az = plsc.load_gather(pzb, [ri]) - plsc.load_gather(pzb, [cj])
        dxb[sl] = ax
        dyb[sl] = ay
        dzb[sl] = az
        ssb[sl] = ax * ax + ay * ay + az * az
        return carry

    lax.fori_loop(0, EW // 16, grp, 0)
    pltpu.sync_copy(dxb, dx_hbm.at[pl.ds(base, EW)])
    pltpu.sync_copy(dyb, dy_hbm.at[pl.ds(base, EW)])
    pltpu.sync_copy(dzb, dz_hbm.at[pl.ds(base, EW)])
    pltpu.sync_copy(ssb, ss_hbm.at[pl.ds(base, EW)])


@functools.lru_cache(maxsize=None)
def _build_geom_sc():
    return pl.kernel(
        _geom_sc_body,
        out_type=(
            jax.ShapeDtypeStruct((N_EDGES,), F32),
            jax.ShapeDtypeStruct((N_EDGES,), F32),
            jax.ShapeDtypeStruct((N_EDGES,), F32),
            jax.ShapeDtypeStruct((N_EDGES,), F32),
        ),
        mesh=_mesh(),
        scratch_types=[
            pltpu.VMEM((N_NODES,), F32),
            pltpu.VMEM((N_NODES,), F32),
            pltpu.VMEM((N_NODES,), F32),
            pltpu.VMEM((EW,), I32),
            pltpu.VMEM((EW,), I32),
            pltpu.VMEM((EW,), F32),
            pltpu.VMEM((EW,), F32),
            pltpu.VMEM((EW,), F32),
            pltpu.VMEM((EW,), F32),
        ],
        compiler_params=pltpu.CompilerParams(needs_layout_passes=False),
    )


def _geom_sc(px, py, pz, row, col):
    return _build_geom_sc()(px, py, pz, row, col)


def _msg_fwd_sc_body(W_hbm, h1_hbm, row_hbm, col_hbm, z_hbm, out_hbm,
                     rb0, cb0, wb0, hb0, rb1, cb1, wb1, hb1, agg_sp,
                     sem0, sem1):
    c = lax.axis_index("c")
    s = lax.axis_index("s")
    wid = c * NS + s
    pltpu.sync_copy(z_hbm, agg_sp.at[pl.ds(s * NPT, NPT)])
    plsc.subcore_barrier()
    base = wid * EW

    def start(j, rb, cb, wb, hb, sem):
        e0 = base + j * KM
        pltpu.sync_copy(col_hbm.at[pl.ds(e0, KM)], cb)
        pltpu.async_copy(h1_hbm.at[cb], hb, sem)
        pltpu.async_copy(row_hbm.at[pl.ds(e0, KM)], rb, sem)
        pltpu.async_copy(W_hbm.at[pl.ds(e0, KM)], wb, sem)

    def finish(rb, cb, wb, hb, sem):
        pltpu.make_async_copy(h1_hbm.at[pl.ds(base, KM)], hb, sem).wait()
        pltpu.make_async_copy(row_hbm.at[pl.ds(base, KM)], rb, sem).wait()
        pltpu.make_async_copy(W_hbm.at[pl.ds(base, KM)], wb, sem).wait()

        def rowloop(r, carry2):
            for f in range(HIDDEN // 16):
                sl = pl.ds(f * 16, 16)
                hb[r, sl] = wb[r, sl] * hb[r, sl]
            return carry2

        lax.fori_loop(0, KM, rowloop, 0)
        pltpu.sync_copy(hb, agg_sp.at[rb], add=True)

    start(0, rb0, cb0, wb0, hb0, sem0)

    def pairloop(t, carry):
        j0 = 2 * t
        start(j0 + 1, rb1, cb1, wb1, hb1, sem1)
        finish(rb0, cb0, wb0, hb0, sem0)

        @pl.when(j0 + 2 < CM)
        def _():
            start(j0 + 2, rb0, cb0, wb0, hb0, sem0)

        finish(rb1, cb1, wb1, hb1, sem1)
        return carry

    lax.fori_loop(0, CM // 2, pairloop, 0)
    if CM % 2 == 1:
        finish(rb0, cb0, wb0, hb0, sem0)
    plsc.subcore_barrier()
    pltpu.sync_copy(agg_sp.at[pl.ds(s * NPT, NPT)],
                    out_hbm.at[c, pl.ds(s * NPT, NPT)])


@functools.lru_cache(maxsize=None)
def _build_msg_fwd_sc():
    return pl.kernel(
        _msg_fwd_sc_body,
        out_type=jax.ShapeDtypeStruct((NC, NPAD, HIDDEN), F32),
        mesh=_mesh(),
        scratch_types=[
            pltpu.VMEM((KM,), I32),
            pltpu.VMEM((KM,), I32),
            pltpu.VMEM((KM, HIDDEN), F32),
            pltpu.VMEM((KM, HIDDEN), F32),
            pltpu.VMEM((KM,), I32),
            pltpu.VMEM((KM,), I32),
            pltpu.VMEM((KM, HIDDEN), F32),
            pltpu.VMEM((KM, HIDDEN), F32),
            pltpu.VMEM_SHARED((NPAD, HIDDEN), F32),
            pltpu.SemaphoreType.DMA,
            pltpu.SemaphoreType.DMA,
        ],
        compiler_params=pltpu.CompilerParams(needs_layout_passes=False),
    )


def _msg_fwd_sc(W, h1, row, col, z):
    return _build_msg_fwd_sc()(W, h1, row, col, z)


def _msg_bwd_sc_body(W_hbm, h1_hbm, gagg_hbm, row_hbm, col_hbm, z_hbm,
                     gw_hbm, out_hbm, rb0, cb0, wb0, hb0, gb0, rb1, cb1,
                     wb1, hb1, gb1, gh1_sp, sem0, sem1):
    c = lax.axis_index("c")
    s = lax.axis_index("s")
    wid = c * NS + s
    pltpu.sync_copy(z_hbm, gh1_sp.at[pl.ds(s * NPT, NPT)])
    plsc.subcore_barrier()
    base = wid * EW

    def start(j, rb, cb, wb, hb, gb, sem):
        e0 = base + j * KB
        pltpu.sync_copy(row_hbm.at[pl.ds(e0, KB)], rb)
        pltpu.sync_copy(col_hbm.at[pl.ds(e0, KB)], cb)
        pltpu.async_copy(gagg_hbm.at[rb], gb, sem)
        pltpu.async_copy(h1_hbm.at[cb], hb, sem)
        pltpu.async_copy(W_hbm.at[pl.ds(e0, KB)], wb, sem)

    def finish(j, rb, cb, wb, hb, gb, sem):
        e0 = base + j * KB
        pltpu.make_async_copy(gagg_hbm.at[pl.ds(0, KB)], gb, sem).wait()
        pltpu.make_async_copy(h1_hbm.at[pl.ds(0, KB)], hb, sem).wait()
        pltpu.make_async_copy(W_hbm.at[pl.ds(base, KB)], wb, sem).wait()

        def rowloop(r, carry2):
            for f in range(HIDDEN // 16):
                sl = pl.ds(f * 16, 16)
                g16 = gb[r, sl]
                wb[r, sl] = g16 * wb[r, sl]
                gb[r, sl] = g16 * hb[r, sl]
            return carry2

        lax.fori_loop(0, KB, rowloop, 0)
        pltpu.sync_copy(wb, gh1_sp.at[cb], add=True)
        pltpu.sync_copy(gb, gw_hbm.at[pl.ds(e0, KB)])

    start(0, rb0, cb0, wb0, hb0, gb0, sem0)

    def pairloop(t, carry):
        j0 = 2 * t
        start(j0 + 1, rb1, cb1, wb1, hb1, gb1, sem1)
        finish(j0, rb0, cb0, wb0, hb0, gb0, sem0)

        @pl.when(j0 + 2 < CB)
        def _():
            start(j0 + 2, rb0, cb0, wb0, hb0, gb0, sem0)

        finish(j0 + 1, rb1, cb1, wb1, hb1, gb1, sem1)
        return carry

    lax.fori_loop(0, CB // 2, pairloop, 0)
    if CB % 2 == 1:
        finish(CB - 1, rb0, cb0, wb0, hb0, gb0, sem0)
    plsc.subcore_barrier()
    pltpu.sync_copy(gh1_sp.at[pl.ds(s * NPT, NPT)],
                    out_hbm.at[c, pl.ds(s * NPT, NPT)])


@functools.lru_cache(maxsize=None)
def _build_msg_bwd_sc():
    return pl.kernel(
        _msg_bwd_sc_body,
        out_type=(
            jax.ShapeDtypeStruct((N_EDGES, HIDDEN), F32),
            jax.ShapeDtypeStruct((NC, NPAD, HIDDEN), F32),
        ),
        mesh=_mesh(),
        scratch_types=[
            pltpu.VMEM((KB,), I32),
            pltpu.VMEM((KB,), I32),
            pltpu.VMEM((KB, HIDDEN), F32),
            pltpu.VMEM((KB, HIDDEN), F32),
            pltpu.VMEM((KB, HIDDEN), F32),
            pltpu.VMEM((KB,), I32),
            pltpu.VMEM((KB,), I32),
            pltpu.VMEM((KB, HIDDEN), F32),
            pltpu.VMEM((KB, HIDDEN), F32),
            pltpu.VMEM((KB, HIDDEN), F32),
            pltpu.VMEM_SHARED((NPAD, HIDDEN), F32),
            pltpu.SemaphoreType.DMA,
            pltpu.SemaphoreType.DMA,
        ],
        compiler_params=pltpu.CompilerParams(needs_layout_passes=False),
    )


def _msg_bwd_sc(W, h1, gagg, row, col, z):
    return _build_msg_bwd_sc()(W, h1, gagg, row, col, z)


def _force_sc_body(dx_hbm, dy_hbm, dz_hbm, gs_hbm, row_hbm, col_hbm,
                   out_hbm, rb0, cb0, dx0, dy0, dz0, gs0, sb0, nb0,
                   rb1, cb1, dx1, dy1, dz1, gs1b, sb1, nb1, fsp,
                   sem0, sem1):
    c = lax.axis_index("c")
    s = lax.axis_index("s")
    wid = c * NS + s
    base = wid * EW

    zero16 = jnp.zeros((16,), F32)

    def zrow(r, carry):
        for f in range(HIDDEN // 16):
            sl = pl.ds(f * 16, 16)
            sb0[r, sl] = zero16
            nb0[r, sl] = zero16
            sb1[r, sl] = zero16
            nb1[r, sl] = zero16
        return carry

    lax.fori_loop(0, KM, zrow, 0)

    def ztab(q, carry):
        pltpu.sync_copy(sb0, fsp.at[pl.ds(s * NPT + q * KM, KM)])
        return carry

    lax.fori_loop(0, NPT // KM, ztab, 0)
    plsc.subcore_barrier()

    lanes = lax.iota(I32, 16)
    col0 = jnp.zeros((16,), I32)
    col1 = col0 + 1
    col2 = col0 + 2

    def start(ci, rb, cb, dxb, dyb, dzb, gsb, sem):
        e0 = base + ci * KM
        pltpu.async_copy(row_hbm.at[pl.ds(e0, KM)], rb, sem)
        pltpu.async_copy(col_hbm.at[pl.ds(e0, KM)], cb, sem)
        pltpu.async_copy(dx_hbm.at[pl.ds(e0, KM)], dxb, sem)
        pltpu.async_copy(dy_hbm.at[pl.ds(e0, KM)], dyb, sem)
        pltpu.async_copy(dz_hbm.at[pl.ds(e0, KM)], dzb, sem)
        pltpu.async_copy(gs_hbm.at[pl.ds(e0, KM)], gsb, sem)

    def finish(rb, cb, dxb, dyb, dzb, gsb, sb, nb, sem):
        pltpu.make_async_copy(row_hbm.at[pl.ds(base, KM)], rb, sem).wait()
        pltpu.make_async_copy(col_hbm.at[pl.ds(base, KM)], cb, sem).wait()
        pltpu.make_async_copy(dx_hbm.at[pl.ds(base, KM)], dxb, sem).wait()
        pltpu.make_async_copy(dy_hbm.at[pl.ds(base, KM)], dyb, sem).wait()
        pltpu.make_async_copy(dz_hbm.at[pl.ds(base, KM)], dzb, sem).wait()
        pltpu.make_async_copy(gs_hbm.at[pl.ds(base, KM)], gsb, sem).wait()

        def grp(g, carry2):
            sl = pl.ds(g * 16, 16)
            gsv = gsb[sl]
            vx = dxb[sl] * gsv
            vy = dyb[sl] * gsv
            vz = dzb[sl] * gsv
            rows = lanes + g * 16
            plsc.store_scatter(sb, [rows, col0], vx)
            plsc.store_scatter(sb, [rows, col1], vy)
            plsc.store_scatter(sb, [rows, col2], vz)
            plsc.store_scatter(nb, [rows, col0], -vx)
            plsc.store_scatter(nb, [rows, col1], -vy)
            plsc.store_scatter(nb, [rows, col2], -vz)
            return carry2

        lax.fori_loop(0, KM // 16, grp, 0)
        pltpu.sync_copy(sb, fsp.at[rb], add=True)
        pltpu.sync_copy(nb, fsp.at[cb], add=True)

    start(0, rb0, cb0, dx0, dy0, dz0, gs0, sem0)

    def pairloop(t, carry):
        j0 = 2 * t
        start(j0 + 1, rb1, cb1, dx1, dy1, dz1, gs1b, sem1)
        finish(rb0, cb0, dx0, dy0, dz0, gs0, sb0, nb0, sem0)

        @pl.when(j0 + 2 < CM)
        def _():
            start(j0 + 2, rb0, cb0, dx0, dy0, dz0, gs0, sem0)

        finish(rb1, cb1, dx1, dy1, dz1, gs1b, sb1, nb1, sem1)
        return carry

    lax.fori_loop(0, CM // 2, pairloop, 0)
    if CM % 2 == 1:
        finish(rb0, cb0, dx0, dy0, dz0, gs0, sb0, nb0, sem0)
    plsc.subcore_barrier()
    pltpu.sync_copy(fsp.at[pl.ds(s * NPT, NPT)],
                    out_hbm.at[c, pl.ds(s * NPT, NPT)])


@functools.lru_cache(maxsize=None)
def _build_force_sc():
    return pl.kernel(
        _force_sc_body,
        out_type=jax.ShapeDtypeStruct((NC, NPAD, HIDDEN), F32),
        mesh=_mesh(),
        scratch_types=[
            pltpu.VMEM((KM,), I32),
            pltpu.VMEM((KM,), I32),
            pltpu.VMEM((KM,), F32),
            pltpu.VMEM((KM,), F32),
            pltpu.VMEM((KM,), F32),
            pltpu.VMEM((KM,), F32),
            pltpu.VMEM((KM, HIDDEN), F32),
            pltpu.VMEM((KM, HIDDEN), F32),
            pltpu.VMEM((KM,), I32),
            pltpu.VMEM((KM,), I32),
            pltpu.VMEM((KM,), F32),
            pltpu.VMEM((KM,), F32),
            pltpu.VMEM((KM,), F32),
            pltpu.VMEM((KM,), F32),
            pltpu.VMEM((KM, HIDDEN), F32),
            pltpu.VMEM((KM, HIDDEN), F32),
            pltpu.VMEM_SHARED((NPAD, HIDDEN), F32),
            pltpu.SemaphoreType.DMA,
            pltpu.SemaphoreType.DMA,
        ],
        compiler_params=pltpu.CompilerParams(needs_layout_passes=False),
    )


def _force_sc(dx, dy, dz, gs, row, col):
    return _build_force_sc()(dx, dy, dz, gs, row, col)


def kernel(atomic_numbers, positions, edge_index, structure_index, params):
    inter = params['interactions']
    row = edge_index[0].astype(I32)
    col = edge_index[1].astype(I32)
    px = positions[:, 0]
    py = positions[:, 1]
    pz = positions[:, 2]
    an2 = atomic_numbers.astype(I32).reshape(N_NODES, 1)
    seg2 = structure_index.astype(I32).reshape(N_NODES, 1)
    z128 = jnp.zeros((NPT, HIDDEN), F32)

    b1 = [p['mlp_b1'].reshape(1, HIDDEN) for p in inter]
    b2 = [p['mlp_b2'].reshape(1, HIDDEN) for p in inter]
    l2b = [p['lin2_b'].reshape(1, HIDDEN) for p in inter]
    lnb = [p['lin_b'].reshape(1, HIDDEN) for p in inter]

    offsets = jnp.linspace(0.0, CUTOFF, NGAUSS)
    gamma = 0.5 / (offsets[1] - offsets[0]) ** 2
    off_in = offsets.reshape(1, NGAUSS)
    ng_in = (-gamma).reshape(1, 1)

    dx, dy, dz, sumsq = _geom_sc(px, py, pz, row, col)
    ss2 = sumsq.reshape(N_EDGES, 1)

    h1 = _node0(an2, params['emb'], inter[0]['lin1_w'])
    Ws = []
    h1s = []
    ts = []
    h = None
    for i in range(NINT):
        p = inter[i]
        W = _edge_mlp(ss2, off_in, ng_in, p['mlp_w1'], b1[i],
                      p['mlp_w2'], b2[i])
        parts = _msg_fwd_sc(W, h1, row, col, z128)
        t, h = _node_t(parts, p['lin2_w'], l2b[i], p['lin_w'], lnb[i])
        Ws.append(W)
        h1s.append(h1)
        ts.append(t)
        if i + 1 < NINT:
            h1 = _mm(h, inter[i + 1]['lin1_w'])

    en64, st64 = _readout(h, seg2, params['energy_w'],
                          params['energy_b'].reshape(1, 1),
                          params['stress_w'],
                          params['stress_b'].reshape(1, 6))
    energy = en64.reshape(NSEG)
    stress = st64

    g_h = jnp.broadcast_to(params['energy_w'][:, 0][None, :],
                           (N_NODES, HIDDEN))
    gacc = jnp.zeros((N_EDGES, 1), F32)
    for i in reversed(range(NINT)):
        p = inter[i]
        g_agg = _node_bwd(g_h, ts[i], p['lin_w'], p['lin2_w'])
        gW, gh1_parts = _msg_bwd_sc(Ws[i], h1s[i], g_agg, row, col, z128)
        gacc = _edge_bwd(ss2, off_in, ng_in, gW, p['mlp_w1'], b1[i],
                         p['mlp_w2'], b2[i], gacc, final=(i == 0))
        if i > 0:
            g_h = _node_bwd_h1(gh1_parts, p['lin1_w'])

    gs1 = gacc.reshape(N_EDGES)
    fparts = _force_sc(dx, dy, dz, gs1, row, col)
    forces = _force_fin(fparts)
    return forces, energy, stress
